# Initial kernel scaffold; baseline (speedup 1.0000x reference)
import jax
import jax.numpy as jnp
from jax.experimental import pallas as pl


def kernel(item_id, text_tokens, image_embedding, item_table, text_table, W_img, b_img, W_proj, b_proj):
    raise NotImplementedError("write your pallas kernel here")


# speedup vs baseline: 3.4921x; 3.4921x over previous
import functools

import jax
import jax.numpy as jnp
from jax import lax
from jax.experimental import pallas as pl
from jax.experimental.pallas import tpu as pltpu
from jax.experimental.pallas import tpu_sc as plsc

B = 4096
EMB = 64
SEQ = 32
IMG_D = 512

NC = 2
NS = 16
NW = NC * NS
BPW = B // NW
TOK_PER_W = BPW * SEQ
CHUNK = 128
NCH = TOK_PER_W // CHUNK
ROWS_PER_CHUNK = CHUNK // SEQ
NL = EMB // 16


def _sc_gather_impl(item_id, tok2d, item_table, text_table):
  mesh = plsc.VectorSubcoreMesh(core_axis_name="c", subcore_axis_name="s")

  @functools.partial(
      pl.kernel,
      mesh=mesh,
      compiler_params=pltpu.CompilerParams(use_tc_tiling_on_sc=False),
      out_type=[
          jax.ShapeDtypeStruct((B, EMB), jnp.float32),
          jax.ShapeDtypeStruct((B, EMB), jnp.float32),
      ],
      scratch_types=[
          pltpu.VMEM((BPW,), jnp.int32),
          pltpu.VMEM((BPW, EMB), jnp.float32),
          pltpu.VMEM((NCH, CHUNK), jnp.int32),
          pltpu.VMEM((CHUNK, EMB), jnp.float32),
          pltpu.VMEM((BPW, EMB), jnp.float32),
          pltpu.SemaphoreType.DMA,
          pltpu.SemaphoreType.DMA,
      ],
  )
  def sc_kernel(item_id_hbm, tok_hbm, item_table_hbm, text_table_hbm,
                item_out_hbm, text_sum_hbm,
                item_idx_v, item_rows_v, tok_idx_v, gbuf, acc,
                sem_item, sem_text):
    wid = lax.axis_index("s") * NC + lax.axis_index("c")
    base = wid * BPW

    pltpu.sync_copy(item_id_hbm.at[pl.ds(base, BPW)], item_idx_v)
    item_cp = pltpu.async_copy(item_table_hbm.at[item_idx_v], item_rows_v,
                               sem_item)

    pltpu.sync_copy(tok_hbm.at[pl.ds(wid * NCH, NCH)], tok_idx_v)

    @pl.loop(0, NCH)
    def chunk_loop(c):
      cp = pltpu.async_copy(text_table_hbm.at[tok_idx_v.at[c]], gbuf,
                            sem_text)
      cp.wait()
      for i in range(ROWS_PER_CHUNK):
        accs = [None] * NL
        for j in range(SEQ):
          r = i * SEQ + j
          for l in range(NL):
            v = gbuf[r, pl.ds(l * 16, 16)]
            accs[l] = v if accs[l] is None else accs[l] + v
        row = c * ROWS_PER_CHUNK + i
        for l in range(NL):
          acc[row, pl.ds(l * 16, 16)] = accs[l]

    item_cp.wait()
    pltpu.sync_copy(item_rows_v, item_out_hbm.at[pl.ds(base, BPW)])
    pltpu.sync_copy(acc, text_sum_hbm.at[pl.ds(base, BPW)])

  return sc_kernel(item_id, tok2d, item_table, text_table)


def _tc_body(item_ref, text_ref, img_ref, wimg_ref, bimg_ref, wproj_ref,
             bproj_ref, out_ref):
  img_vec = jnp.dot(img_ref[...], wimg_ref[...],
                    preferred_element_type=jnp.float32) + bimg_ref[...]
  h = jnp.concatenate(
      [item_ref[...], text_ref[...] * (1.0 / SEQ), img_vec], axis=-1)
  out_ref[...] = jnp.dot(h, wproj_ref[...],
                         preferred_element_type=jnp.float32) + bproj_ref[...]


def _tc_proj(item_vec, text_sum, image_embedding, W_img, b_img, W_proj,
             b_proj):
  TILE = 512
  grid = (B // TILE,)
  return pl.pallas_call(
      _tc_body,
      grid=grid,
      in_specs=[
          pl.BlockSpec((TILE, EMB), lambda i: (i, 0)),
          pl.BlockSpec((TILE, EMB), lambda i: (i, 0)),
          pl.BlockSpec((TILE, IMG_D), lambda i: (i, 0)),
          pl.BlockSpec((IMG_D, EMB), lambda i: (0, 0)),
          pl.BlockSpec((1, EMB), lambda i: (0, 0)),
          pl.BlockSpec((3 * EMB, EMB), lambda i: (0, 0)),
          pl.BlockSpec((1, EMB), lambda i: (0, 0)),
      ],
      out_specs=pl.BlockSpec((TILE, EMB), lambda i: (i, 0)),
      out_shape=jax.ShapeDtypeStruct((B, EMB), jnp.float32),
  )(item_vec, text_sum, image_embedding, W_img, b_img, W_proj, b_proj)


@jax.jit
def kernel(item_id, text_tokens, image_embedding, item_table, text_table,
           W_img, b_img, W_proj, b_proj):
  tok2d = text_tokens.reshape(B * SEQ // CHUNK, CHUNK)
  item_vec, text_sum = _sc_gather_impl(item_id, tok2d, item_table, text_table)
  return _tc_proj(item_vec, text_sum, image_embedding, W_img,
                  b_img.reshape(1, EMB), W_proj, b_proj.reshape(1, EMB))


# speedup vs baseline: 4.4171x; 1.2649x over previous
import functools

import jax
import jax.numpy as jnp
from jax import lax
from jax.experimental import pallas as pl
from jax.experimental.pallas import tpu as pltpu
from jax.experimental.pallas import tpu_sc as plsc

B = 4096
EMB = 64
SEQ = 32
IMG_D = 512
N_ITEMS = 100001

NC = 2
NS = 16
NW = NC * NS
BPW = B // NW
TOK_PER_W = BPW * SEQ
CHUNK = 128
NCH = TOK_PER_W // CHUNK
ROWS_PER_CHUNK = CHUNK // SEQ
NL = EMB // 16
ITEM_PAD_ROWS = ((N_ITEMS + 7) // 8) * 8


def _sc_gather_impl(item_id, tok2d, item_pad, text_table):
  mesh = plsc.VectorSubcoreMesh(core_axis_name="c", subcore_axis_name="s")

  @functools.partial(
      pl.kernel,
      mesh=mesh,
      compiler_params=pltpu.CompilerParams(use_tc_tiling_on_sc=False),
      out_type=[
          jax.ShapeDtypeStruct((B, 2 * EMB), jnp.float32),
          jax.ShapeDtypeStruct((B, 2 * EMB), jnp.float32),
      ],
      scratch_types=[
          pltpu.VMEM((BPW,), jnp.int32),
          pltpu.VMEM((BPW, 2 * EMB), jnp.float32),
          pltpu.VMEM((NCH, CHUNK), jnp.int32),
          pltpu.VMEM((CHUNK, EMB), jnp.float32),
          pltpu.VMEM((CHUNK, EMB), jnp.float32),
          pltpu.VMEM((BPW, 2 * EMB), jnp.float32),
          pltpu.SemaphoreType.DMA,
          pltpu.SemaphoreType.DMA,
          pltpu.SemaphoreType.DMA,
      ],
  )
  def sc_kernel(item_id_hbm, tok_hbm, item_pad_hbm, text_table_hbm,
                item_out_hbm, text_out_hbm,
                item_idx_v, item_rows_v, tok_idx_v, gbuf0, gbuf1, acc,
                sem_item, sem_t0, sem_t1):
    wid = lax.axis_index("s") * NC + lax.axis_index("c")
    base = wid * BPW
    gbufs = (gbuf0, gbuf1)
    sems = (sem_t0, sem_t1)

    pltpu.sync_copy(item_id_hbm.at[pl.ds(base, BPW)], item_idx_v)
    item_cp = pltpu.async_copy(item_pad_hbm.at[item_idx_v], item_rows_v,
                               sem_item)

    pltpu.sync_copy(tok_hbm.at[pl.ds(wid * NCH, NCH)], tok_idx_v)

    pltpu.async_copy(text_table_hbm.at[tok_idx_v.at[0]], gbufs[0], sems[0])

    @pl.loop(0, NCH, step=2)
    def chunk_loop(c):
      for b in range(2):
        cc = c + b
        nxt = (b + 1) % 2

        @pl.when(cc + 1 < NCH)
        def _():
          pltpu.async_copy(text_table_hbm.at[tok_idx_v.at[cc + 1]],
                           gbufs[nxt], sems[nxt])

        pltpu.make_async_copy(text_table_hbm.at[tok_idx_v.at[0]],
                              gbufs[b], sems[b]).wait()
        gbuf = gbufs[b]
        for i in range(ROWS_PER_CHUNK):
          accs = [None] * NL
          for j in range(SEQ):
            r = i * SEQ + j
            for l in range(NL):
              v = gbuf[r, pl.ds(l * 16, 16)]
              accs[l] = v if accs[l] is None else accs[l] + v
          row = cc * ROWS_PER_CHUNK + i
          for l in range(NL):
            acc[row, pl.ds(l * 16, 16)] = accs[l]

    item_cp.wait()
    pltpu.sync_copy(item_rows_v, item_out_hbm.at[pl.ds(base, BPW)])
    pltpu.sync_copy(acc, text_out_hbm.at[pl.ds(base, BPW)])

  return sc_kernel(item_id, tok2d, item_pad, text_table)


def _tc_img_body(img_ref, wimg_ref, bimg_ref, out_ref):
  out_ref[...] = jnp.dot(img_ref[...], wimg_ref[...],
                         preferred_element_type=jnp.float32) + bimg_ref[...]


def _tc_img(image_embedding, W_img, b_img):
  TILE = 512
  return pl.pallas_call(
      _tc_img_body,
      grid=(B // TILE,),
      in_specs=[
          pl.BlockSpec((TILE, IMG_D), lambda i: (i, 0)),
          pl.BlockSpec((IMG_D, EMB), lambda i: (0, 0)),
          pl.BlockSpec((1, EMB), lambda i: (0, 0)),
      ],
      out_specs=pl.BlockSpec((TILE, EMB), lambda i: (i, 0)),
      out_shape=jax.ShapeDtypeStruct((B, EMB), jnp.float32),
  )(image_embedding, W_img, b_img)


def _tc_final_body(item_ref, text_ref, img_ref, wproj_ref, bproj_ref,
                   out_ref):
  w = wproj_ref[...]
  part = jnp.dot(item_ref[:, :EMB], w[:EMB, :],
                 preferred_element_type=jnp.float32)
  part += jnp.dot(text_ref[:, :EMB] * (1.0 / SEQ), w[EMB:2 * EMB, :],
                  preferred_element_type=jnp.float32)
  part += jnp.dot(img_ref[...], w[2 * EMB:, :],
                  preferred_element_type=jnp.float32)
  out_ref[...] = part + bproj_ref[...]


def _tc_final(item_vec, text_sum, img_vec, W_proj, b_proj):
  TILE = 512
  return pl.pallas_call(
      _tc_final_body,
      grid=(B // TILE,),
      in_specs=[
          pl.BlockSpec((TILE, 2 * EMB), lambda i: (i, 0)),
          pl.BlockSpec((TILE, 2 * EMB), lambda i: (i, 0)),
          pl.BlockSpec((TILE, EMB), lambda i: (i, 0)),
          pl.BlockSpec((3 * EMB, EMB), lambda i: (0, 0)),
          pl.BlockSpec((1, EMB), lambda i: (0, 0)),
      ],
      out_specs=pl.BlockSpec((TILE, EMB), lambda i: (i, 0)),
      out_shape=jax.ShapeDtypeStruct((B, EMB), jnp.float32),
  )(item_vec, text_sum, img_vec, W_proj, b_proj)


@jax.jit
def kernel(item_id, text_tokens, image_embedding, item_table, text_table,
           W_img, b_img, W_proj, b_proj):
  tok2d = text_tokens.reshape(B * SEQ // CHUNK, CHUNK)
  item_pad = jnp.pad(item_table,
                     ((0, ITEM_PAD_ROWS - N_ITEMS), (0, 2 * EMB - EMB)))
  img_vec = _tc_img(image_embedding, W_img, b_img.reshape(1, EMB))
  item_vec, text_sum = _sc_gather_impl(item_id, tok2d, item_pad, text_table)
  return _tc_final(item_vec, text_sum, img_vec, W_proj,
                   b_proj.reshape(1, EMB))


# speedup vs baseline: 4.6562x; 1.0541x over previous
import functools

import jax
import jax.numpy as jnp
from jax import lax
from jax.experimental import pallas as pl
from jax.experimental.pallas import tpu as pltpu
from jax.experimental.pallas import tpu_sc as plsc

B = 4096
EMB = 64
SEQ = 32
IMG_D = 512
N_ITEMS = 100001
N_TOKENS = 20000

NC = 2
NS = 16
NW = NC * NS
BPW = B // NW
TOK_PER_W = BPW * SEQ
CHUNK = 128
NCH = TOK_PER_W // CHUNK
ROWS_PER_CHUNK = CHUNK // SEQ
NL = EMB // 16

PACK_BLK = 1024


def _pack_body(left_ref, right_ref, out_ref):
  out_ref[...] = jnp.concatenate([left_ref[...].T, right_ref[...].T], axis=1)


def _pack_table(table, n_rows):
  nblk = (n_rows + PACK_BLK - 1) // PACK_BLK
  assert nblk % 2 == 0, "even block count required"
  pad = nblk * PACK_BLK
  half_blk = nblk // 2
  t_t = table.T
  out = pl.pallas_call(
      _pack_body,
      grid=(half_blk,),
      in_specs=[
          pl.BlockSpec((EMB, PACK_BLK), lambda i: (0, i)),
          pl.BlockSpec((EMB, PACK_BLK), lambda i: (0, half_blk + i)),
      ],
      out_specs=pl.BlockSpec((PACK_BLK, 2 * EMB), lambda i: (i, 0)),
      out_shape=jax.ShapeDtypeStruct((pad // 2, 2 * EMB), jnp.float32),
  )(t_t, t_t)
  return out, pad // 2


def _remap_idx(idx, half):
  return jnp.where(idx < half, 2 * idx, 2 * (idx - half) + 1).astype(jnp.int32)


def _sc_text(tok2d, text_packed):
  rows2 = text_packed.shape[0] * 2
  text_lin = text_packed.reshape(rows2, EMB)
  mesh = plsc.VectorSubcoreMesh(core_axis_name="c", subcore_axis_name="s")

  @functools.partial(
      pl.kernel,
      mesh=mesh,
      compiler_params=pltpu.CompilerParams(use_tc_tiling_on_sc=False),
      out_type=jax.ShapeDtypeStruct((B, 2 * EMB), jnp.float32),
      scratch_types=[
          pltpu.VMEM((NCH, CHUNK), jnp.int32),
          pltpu.VMEM((CHUNK, EMB), jnp.float32),
          pltpu.VMEM((CHUNK, EMB), jnp.float32),
          pltpu.VMEM((BPW, 2 * EMB), jnp.float32),
          pltpu.SemaphoreType.DMA,
          pltpu.SemaphoreType.DMA,
      ],
  )
  def sc_kernel(tok_hbm, table_hbm, text_out_hbm,
                tok_idx_v, gbuf0, gbuf1, acc, sem_t0, sem_t1):
    wid = lax.axis_index("s") * NC + lax.axis_index("c")
    base = wid * BPW
    gbufs = (gbuf0, gbuf1)
    sems = (sem_t0, sem_t1)

    pltpu.sync_copy(tok_hbm.at[pl.ds(wid * NCH, NCH)], tok_idx_v)
    pltpu.async_copy(table_hbm.at[tok_idx_v.at[0]], gbufs[0], sems[0])

    @pl.loop(0, NCH, step=2)
    def chunk_loop(c):
      for b in range(2):
        cc = c + b
        nxt = (b + 1) % 2

        @pl.when(cc + 1 < NCH)
        def _():
          pltpu.async_copy(table_hbm.at[tok_idx_v.at[cc + 1]],
                           gbufs[nxt], sems[nxt])

        pltpu.make_async_copy(table_hbm.at[tok_idx_v.at[0]],
                              gbufs[b], sems[b]).wait()
        gbuf = gbufs[b]
        for i in range(ROWS_PER_CHUNK):
          accs = [None] * NL
          for j in range(SEQ):
            r = i * SEQ + j
            for l in range(NL):
              v = gbuf[r, pl.ds(l * 16, 16)]
              accs[l] = v if accs[l] is None else accs[l] + v
          row = cc * ROWS_PER_CHUNK + i
          for l in range(NL):
            acc[row, pl.ds(l * 16, 16)] = accs[l]

    pltpu.sync_copy(acc, text_out_hbm.at[pl.ds(base, BPW)])

  return sc_kernel(tok2d, text_lin)


def _sc_item(item_idx, item_packed):
  rows2 = item_packed.shape[0] * 2
  item_lin = item_packed.reshape(rows2, EMB)
  mesh = plsc.VectorSubcoreMesh(core_axis_name="c", subcore_axis_name="s")

  @functools.partial(
      pl.kernel,
      mesh=mesh,
      compiler_params=pltpu.CompilerParams(use_tc_tiling_on_sc=False),
      out_type=jax.ShapeDtypeStruct((B, 2 * EMB), jnp.float32),
      scratch_types=[
          pltpu.VMEM((BPW,), jnp.int32),
          pltpu.VMEM((BPW, EMB), jnp.float32),
          pltpu.VMEM((BPW, 2 * EMB), jnp.float32),
          pltpu.SemaphoreType.DMA,
      ],
  )
  def sc_kernel(idx_hbm, table_hbm, out_hbm, idx_v, rows_v, wide_v, sem):
    wid = lax.axis_index("s") * NC + lax.axis_index("c")
    base = wid * BPW
    pltpu.sync_copy(idx_hbm.at[pl.ds(base, BPW)], idx_v)
    pltpu.async_copy(table_hbm.at[idx_v], rows_v, sem).wait()
    for r in range(BPW):
      for l in range(NL):
        wide_v[r, pl.ds(l * 16, 16)] = rows_v[r, pl.ds(l * 16, 16)]
    pltpu.sync_copy(wide_v, out_hbm.at[pl.ds(base, BPW)])

  return sc_kernel(item_idx, item_lin)


def _tc_img_body(img_ref, wimg_ref, bimg_ref, out_ref):
  out_ref[...] = jnp.dot(img_ref[...], wimg_ref[...],
                         preferred_element_type=jnp.float32) + bimg_ref[...]


def _tc_img(image_embedding, W_img, b_img):
  TILE = 512
  return pl.pallas_call(
      _tc_img_body,
      grid=(B // TILE,),
      in_specs=[
          pl.BlockSpec((TILE, IMG_D), lambda i: (i, 0)),
          pl.BlockSpec((IMG_D, EMB), lambda i: (0, 0)),
          pl.BlockSpec((1, EMB), lambda i: (0, 0)),
      ],
      out_specs=pl.BlockSpec((TILE, EMB), lambda i: (i, 0)),
      out_shape=jax.ShapeDtypeStruct((B, EMB), jnp.float32),
  )(image_embedding, W_img, b_img)


def _tc_final_body(item_ref, text_ref, img_ref, wproj_ref, bproj_ref,
                   out_ref):
  w = wproj_ref[...]
  part = jnp.dot(item_ref[:, :EMB], w[:EMB, :],
                 preferred_element_type=jnp.float32)
  part += jnp.dot(text_ref[:, :EMB] * (1.0 / SEQ), w[EMB:2 * EMB, :],
                  preferred_element_type=jnp.float32)
  part += jnp.dot(img_ref[...], w[2 * EMB:, :],
                  preferred_element_type=jnp.float32)
  out_ref[...] = part + bproj_ref[...]


def _tc_final(item_vec, text_sum, img_vec, W_proj, b_proj):
  TILE = 512
  return pl.pallas_call(
      _tc_final_body,
      grid=(B // TILE,),
      in_specs=[
          pl.BlockSpec((TILE, 2 * EMB), lambda i: (i, 0)),
          pl.BlockSpec((TILE, 2 * EMB), lambda i: (i, 0)),
          pl.BlockSpec((TILE, EMB), lambda i: (i, 0)),
          pl.BlockSpec((3 * EMB, EMB), lambda i: (0, 0)),
          pl.BlockSpec((1, EMB), lambda i: (0, 0)),
      ],
      out_specs=pl.BlockSpec((TILE, EMB), lambda i: (i, 0)),
      out_shape=jax.ShapeDtypeStruct((B, EMB), jnp.float32),
  )(item_vec, text_sum, img_vec, W_proj, b_proj)


@jax.jit
def kernel(item_id, text_tokens, image_embedding, item_table, text_table,
           W_img, b_img, W_proj, b_proj):
  text_packed, text_half = _pack_table(text_table, N_TOKENS)
  tok_mapped = _remap_idx(text_tokens, text_half)
  tok2d = tok_mapped.reshape(B * SEQ // CHUNK, CHUNK)
  text_sum = _sc_text(tok2d, text_packed)

  item_packed, item_half = _pack_table(item_table, N_ITEMS)
  item_idx = _remap_idx(item_id, item_half)
  item_vec = _sc_item(item_idx, item_packed)

  img_vec = _tc_img(image_embedding, W_img, b_img.reshape(1, EMB))
  return _tc_final(item_vec, text_sum, img_vec, W_proj,
                   b_proj.reshape(1, EMB))
